# baseline (speedup 1.0000x reference)
import functools
import math

import jax
import jax.numpy as jnp
from jax import lax
from jax.experimental import pallas as pl
from jax.experimental.pallas import tpu as pltpu
from jax.experimental.pallas import tpu_sc as plsc

D_MODEL = 1024
SCALE = math.sqrt(D_MODEL)

NC = 2
NS = 16
L = 16
NW = NC * NS

B_TOTAL = 4 * 8192
B_PER_W = B_TOTAL // NW
CHUNK = 16
N_CHUNKS = B_PER_W // CHUNK
NIN = 4
NOUT = 2
N_OUTER = N_CHUNKS // NIN


GROUP = 8


def _scale_store_chunk(src, dst, out_hbm, row0, ssem):
    def row(i, _):
        for j in range(D_MODEL // L):
            sl = pl.ds(j * L, L)
            dst[i, sl] = src[i, sl] * SCALE

        @pl.when(i % GROUP == GROUP - 1)
        def _():
            r = pl.multiple_of(i - (GROUP - 1), GROUP)
            pltpu.async_copy(
                dst.at[pl.ds(r, GROUP)],
                out_hbm.at[pl.ds(row0 + r, GROUP)],
                ssem)

        return 0

    lax.fori_loop(0, CHUNK, row, 0)


def _body(ids_hbm, table_hbm, out_hbm, idx_v, in0, in1, in2, in3, ot0, ot1,
          g0, g1, g2, g3, s0, s1):
    wid = lax.axis_index("s") * NC + lax.axis_index("c")
    base = wid * B_PER_W

    ins = (in0, in1, in2, in3)
    outs = (ot0, ot1)
    gsems = (g0, g1, g2, g3)
    ssems = (s0, s1)

    pltpu.sync_copy(ids_hbm.at[wid], idx_v)

    for b in range(NIN):
        pltpu.async_copy(table_hbm.at[idx_v.at[b]], ins[b], gsems[b])

    def outer(g, _):
        for b in range(NIN):
            j = g * NIN + b
            ob = b % NOUT
            pltpu.make_async_copy(
                table_hbm.at[pl.ds(0, CHUNK)], ins[b], gsems[b]).wait()

            if b >= NOUT:
                pltpu.make_async_copy(
                    outs[ob], out_hbm.at[pl.ds(base, CHUNK)], ssems[ob]).wait()
            else:
                @pl.when(g > 0)
                def _():
                    pltpu.make_async_copy(
                        outs[ob], out_hbm.at[pl.ds(base, CHUNK)],
                        ssems[ob]).wait()

            _scale_store_chunk(ins[b], outs[ob], out_hbm,
                               base + j * CHUNK, ssems[ob])

            @pl.when(g < N_OUTER - 1)
            def _():
                pltpu.async_copy(
                    table_hbm.at[idx_v.at[j + NIN]], ins[b], gsems[b])

        return 0

    lax.fori_loop(0, N_OUTER, outer, 0)

    for b in range(NOUT):
        pltpu.make_async_copy(
            outs[b], out_hbm.at[pl.ds(base, CHUNK)], ssems[b]).wait()


@jax.jit
def _gather_scale(ids3, table):
    mesh = plsc.VectorSubcoreMesh(core_axis_name="c", subcore_axis_name="s")
    f = functools.partial(
        pl.kernel,
        out_type=jax.ShapeDtypeStruct((B_TOTAL, D_MODEL), jnp.float32),
        mesh=mesh,
        scratch_types=[
            pltpu.VMEM((N_CHUNKS, CHUNK), jnp.int32),
            pltpu.VMEM((CHUNK, D_MODEL), jnp.float32),
            pltpu.VMEM((CHUNK, D_MODEL), jnp.float32),
            pltpu.VMEM((CHUNK, D_MODEL), jnp.float32),
            pltpu.VMEM((CHUNK, D_MODEL), jnp.float32),
            pltpu.VMEM((CHUNK, D_MODEL), jnp.float32),
            pltpu.VMEM((CHUNK, D_MODEL), jnp.float32),
            pltpu.SemaphoreType.DMA,
            pltpu.SemaphoreType.DMA,
            pltpu.SemaphoreType.DMA,
            pltpu.SemaphoreType.DMA,
            pltpu.SemaphoreType.DMA,
            pltpu.SemaphoreType.DMA,
        ],
    )(_body)
    return f(ids3, table)


def kernel(input_ids, table):
    ids3 = input_ids.reshape(NW, N_CHUNKS, CHUNK).astype(jnp.int32)
    out = _gather_scale(ids3, table)
    return out.reshape(*input_ids.shape, D_MODEL)


# speedup vs baseline: 1.0094x; 1.0094x over previous
import functools
import math

import jax
import jax.numpy as jnp
from jax import lax
from jax.experimental import pallas as pl
from jax.experimental.pallas import tpu as pltpu
from jax.experimental.pallas import tpu_sc as plsc

D_MODEL = 1024
SCALE = math.sqrt(D_MODEL)

NC = 2
NS = 16
L = 16
NW = NC * NS

B_TOTAL = 4 * 8192
B_PER_W = B_TOTAL // NW
CHUNK = 16
N_CHUNKS = B_PER_W // CHUNK
NIN = 4
NOUT = 2
N_OUTER = N_CHUNKS // NIN


def _scale_chunk(src, dst):
    def row(i, _):
        for j in range(D_MODEL // L):
            sl = pl.ds(j * L, L)
            dst[i, sl] = src[i, sl] * SCALE
        return 0

    lax.fori_loop(0, CHUNK, row, 0)


def _body(ids_hbm, table_hbm, out_hbm, idx_v, in0, in1, in2, in3, ot0, ot1,
          g0, g1, g2, g3, s0, s1):
    wid = lax.axis_index("s") * NC + lax.axis_index("c")
    base = wid * B_PER_W

    ins = (in0, in1, in2, in3)
    outs = (ot0, ot1)
    gsems = (g0, g1, g2, g3)
    ssems = (s0, s1)

    tiles_per_row = 8192 // B_PER_W
    pltpu.sync_copy(
        ids_hbm.at[wid // tiles_per_row,
                   pl.ds((wid % tiles_per_row) * B_PER_W, B_PER_W)], idx_v)

    def idx_slice(c):
        return idx_v.at[pl.ds(pl.multiple_of(c * CHUNK, CHUNK), CHUNK)]

    for b in range(NIN):
        pltpu.async_copy(table_hbm.at[idx_slice(b)], ins[b], gsems[b])

    def outer(g, _):
        for b in range(NIN):
            j = g * NIN + b
            ob = b % NOUT
            pltpu.make_async_copy(
                table_hbm.at[pl.ds(0, CHUNK)], ins[b], gsems[b]).wait()

            if b >= NOUT:
                pltpu.make_async_copy(
                    outs[ob], out_hbm.at[pl.ds(base, CHUNK)], ssems[ob]).wait()
            else:
                @pl.when(g > 0)
                def _():
                    pltpu.make_async_copy(
                        outs[ob], out_hbm.at[pl.ds(base, CHUNK)],
                        ssems[ob]).wait()

            _scale_chunk(ins[b], outs[ob])

            pltpu.async_copy(
                outs[ob], out_hbm.at[pl.ds(base + j * CHUNK, CHUNK)],
                ssems[ob])

            @pl.when(g < N_OUTER - 1)
            def _():
                pltpu.async_copy(
                    table_hbm.at[idx_slice(j + NIN)], ins[b], gsems[b])

        return 0

    lax.fori_loop(0, N_OUTER, outer, 0)

    for b in range(NOUT):
        pltpu.make_async_copy(
            outs[b], out_hbm.at[pl.ds(base, CHUNK)], ssems[b]).wait()


@jax.jit
def _gather_scale(ids3, table):
    mesh = plsc.VectorSubcoreMesh(core_axis_name="c", subcore_axis_name="s")
    f = functools.partial(
        pl.kernel,
        out_type=jax.ShapeDtypeStruct((B_TOTAL, D_MODEL), jnp.float32),
        mesh=mesh,
        scratch_types=[
            pltpu.VMEM((B_PER_W,), jnp.int32),
            pltpu.VMEM((CHUNK, D_MODEL), jnp.float32),
            pltpu.VMEM((CHUNK, D_MODEL), jnp.float32),
            pltpu.VMEM((CHUNK, D_MODEL), jnp.float32),
            pltpu.VMEM((CHUNK, D_MODEL), jnp.float32),
            pltpu.VMEM((CHUNK, D_MODEL), jnp.float32),
            pltpu.VMEM((CHUNK, D_MODEL), jnp.float32),
            pltpu.SemaphoreType.DMA,
            pltpu.SemaphoreType.DMA,
            pltpu.SemaphoreType.DMA,
            pltpu.SemaphoreType.DMA,
            pltpu.SemaphoreType.DMA,
            pltpu.SemaphoreType.DMA,
        ],
    )(_body)
    return f(ids3, table)


def kernel(input_ids, table):
    out = _gather_scale(input_ids, table)
    return out.reshape(*input_ids.shape, D_MODEL)
